# Initial kernel scaffold; baseline (speedup 1.0000x reference)
import jax
import jax.numpy as jnp
from jax.experimental import pallas as pl


def kernel(x, edge_index, batch, W0a, b0a, W0b, b0b, W1a, b1a, W1b, b1b, W2a, b2a, W2b, b2b, linW, linb):
    raise NotImplementedError("write your pallas kernel here")


# speedup vs baseline: 4.0908x; 4.0908x over previous
import functools

import jax
import jax.numpy as jnp
from jax import lax
from jax.experimental import pallas as pl
from jax.experimental.pallas import tpu as pltpu
from jax.experimental.pallas import tpu_sc as plsc

N = 10000
E = 160000
F_IN = 256
H = 512
C = 40
G = 64

SLAB = 128
NTILE = 16
EPT = E // NTILE
EB = 80
NBATCH = EPT // EB
NPAD = 10240
RPT = NPAD // NTILE


def _make_seg_sum(nslab: int):
    nchunk = nslab // 2
    mesh = plsc.VectorSubcoreMesh(core_axis_name="c", subcore_axis_name="s",
                                  num_cores=2, num_subcores=NTILE)

    @functools.partial(
        pl.kernel,
        out_type=jax.ShapeDtypeStruct((nslab, NPAD, SLAB), jnp.float32),
        mesh=mesh,
        scratch_types=[
            pltpu.VMEM((NBATCH, EB), jnp.int32),
            pltpu.VMEM((NBATCH, EB), jnp.int32),
            pltpu.VMEM((EB, SLAB), jnp.float32),
            pltpu.VMEM_SHARED((NPAD, SLAB), jnp.float32),
            pltpu.SemaphoreType.DMA,
        ],
    )
    def seg(h_hbm, ei_hbm, zeros_hbm, out_hbm, src_v, dst_v, rows_v, agg_sh, sem):
        cid = lax.axis_index("c")
        sid = lax.axis_index("s")
        pltpu.sync_copy(ei_hbm.at[0, sid], src_v)
        pltpu.sync_copy(ei_hbm.at[1, sid], dst_v)
        for c in range(nchunk):
            def offs(k, _):
                for m in range(EB // 16):
                    sl = pl.ds(m * 16, 16)
                    if c == 0:
                        src_v[k, sl] = src_v[k, sl] + cid * N
                    else:
                        src_v[k, sl] = src_v[k, sl] + 2 * N
                return 0
            lax.fori_loop(0, NBATCH, offs, 0)
            rsl = pl.ds(sid * RPT, RPT)
            pltpu.sync_copy(zeros_hbm.at[rsl], agg_sh.at[rsl])
            plsc.subcore_barrier()
            def batch(k, _):
                pltpu.async_copy(h_hbm.at[src_v.at[k]], rows_v, sem).wait()
                pltpu.sync_copy(rows_v, agg_sh.at[dst_v.at[k]], add=True)
                return 0
            lax.fori_loop(0, NBATCH, batch, 0)
            plsc.subcore_barrier()
            pltpu.sync_copy(agg_sh.at[rsl], out_hbm.at[c * 2 + cid, rsl])

    return seg


def _make_mlp(nslab_in: int):
    fin = nslab_in * SLAB
    R = 1000

    def body(h_ref, a_ref, wa_ref, ba_ref, wb_ref, bb_ref, o_ref):
        hh = jnp.concatenate(
            [h_ref[q] + a_ref[q] for q in range(nslab_in)], axis=1)
        t = jnp.dot(hh, wa_ref[...], preferred_element_type=jnp.float32)
        t = jnp.maximum(t + ba_ref[...], 0.0)
        o = jnp.dot(t, wb_ref[...], preferred_element_type=jnp.float32)
        o = jnp.maximum(o + bb_ref[...], 0.0)
        for q in range(4):
            o_ref[q] = o[:, q * SLAB:(q + 1) * SLAB]

    return pl.pallas_call(
        body,
        grid=(N // R,),
        in_specs=[
            pl.BlockSpec((nslab_in, R, SLAB), lambda i: (0, i, 0)),
            pl.BlockSpec((nslab_in, R, SLAB), lambda i: (0, i, 0)),
            pl.BlockSpec((fin, H), lambda i: (0, 0)),
            pl.BlockSpec((1, H), lambda i: (0, 0)),
            pl.BlockSpec((H, H), lambda i: (0, 0)),
            pl.BlockSpec((1, H), lambda i: (0, 0)),
        ],
        out_specs=pl.BlockSpec((4, R, SLAB), lambda i: (0, i, 0)),
        out_shape=jax.ShapeDtypeStruct((4, N, SLAB), jnp.float32),
    )


_RP = 1000


def _pool_body(b_ref, h_ref, w_ref, lb_ref, o_ref, sums_ref, cnt_ref):
    i = pl.program_id(0)

    @pl.when(i == 0)
    def _init():
        sums_ref[...] = jnp.zeros_like(sums_ref)
        cnt_ref[...] = jnp.zeros_like(cnt_ref)

    b = b_ref[0, 0, :]
    oh = (lax.broadcasted_iota(jnp.int32, (G, _RP), 0) == b[None, :]).astype(
        jnp.float32)
    h = jnp.concatenate([h_ref[q] for q in range(4)], axis=1)
    sums_ref[...] += jnp.dot(oh, h, preferred_element_type=jnp.float32)
    cnt_ref[...] += jnp.dot(oh, jnp.ones((_RP, 128), jnp.float32),
                            preferred_element_type=jnp.float32)

    @pl.when(i == pl.num_programs(0) - 1)
    def _fin():
        cnt = jnp.maximum(cnt_ref[:, 0:1], 1.0)
        pooled = sums_ref[...] / cnt
        o_ref[...] = jnp.dot(pooled, w_ref[...],
                             preferred_element_type=jnp.float32) + lb_ref[...]


_POOL = pl.pallas_call(
    _pool_body,
    grid=(N // _RP,),
    in_specs=[
        pl.BlockSpec((1, 1, _RP), lambda i: (i, 0, 0)),
        pl.BlockSpec((4, _RP, SLAB), lambda i: (0, i, 0)),
        pl.BlockSpec((H, 128), lambda i: (0, 0)),
        pl.BlockSpec((1, 128), lambda i: (0, 0)),
    ],
    out_specs=pl.BlockSpec((G, 128), lambda i: (0, 0)),
    out_shape=jax.ShapeDtypeStruct((G, 128), jnp.float32),
    scratch_shapes=[
        pltpu.VMEM((G, H), jnp.float32),
        pltpu.VMEM((G, 128), jnp.float32),
    ],
)

_SEG2 = _make_seg_sum(2)
_SEG4 = _make_seg_sum(4)
_MLP0 = _make_mlp(2)
_MLP1 = _make_mlp(4)


def kernel(x, edge_index, batch, W0a, b0a, W0b, b0b, W1a, b1a, W1b, b1b,
           W2a, b2a, W2b, b2b, linW, linb):
    zeros = jnp.zeros((NPAD, SLAB), jnp.float32)
    ei = edge_index.reshape(2, NTILE, NBATCH, EB)
    x_sm = x.reshape(N, 2, SLAB).transpose(1, 0, 2)
    agg0 = _SEG2(x_sm.reshape(2 * N, SLAB), ei, zeros)
    h1 = _MLP0(x_sm, agg0, W0a, b0a.reshape(1, H), W0b, b0b.reshape(1, H))
    agg1 = _SEG4(h1.reshape(4 * N, SLAB), ei, zeros)
    h2 = _MLP1(h1, agg1, W1a, b1a.reshape(1, H), W1b, b1b.reshape(1, H))
    agg2 = _SEG4(h2.reshape(4 * N, SLAB), ei, zeros)
    h3 = _MLP1(h2, agg2, W2a, b2a.reshape(1, H), W2b, b2b.reshape(1, H))
    wpad = jnp.zeros((H, 128), jnp.float32).at[:, :C].set(linW)
    bpad = jnp.zeros((1, 128), jnp.float32).at[0, :C].set(linb)
    out = _POOL(batch.reshape(N // _RP, 1, _RP), h3, wpad, bpad)
    return out[:, :C]


# speedup vs baseline: 6.5346x; 1.5974x over previous
import functools

import jax
import jax.numpy as jnp
from jax import lax
from jax.experimental import pallas as pl
from jax.experimental.pallas import tpu as pltpu
from jax.experimental.pallas import tpu_sc as plsc

N = 10000
E = 160000
F_IN = 256
H = 512
C = 40
G = 64

SLAB = 128
NTILE = 16
EB = 80
E_PAD = 163840
EPT = E_PAD // NTILE
NBATCH = EPT // EB
NPAD = 10240
RPT = NPAD // NTILE


def _make_seg_sum(nslab: int):
    nchunk = nslab // 2
    mesh = plsc.VectorSubcoreMesh(core_axis_name="c", subcore_axis_name="s",
                                  num_cores=2, num_subcores=NTILE)

    @functools.partial(
        pl.kernel,
        out_type=jax.ShapeDtypeStruct((nslab, NPAD, SLAB), jnp.float32),
        mesh=mesh,
        scratch_types=[
            pltpu.VMEM((EPT,), jnp.int32),
            pltpu.VMEM((NBATCH, EB), jnp.int32),
            pltpu.VMEM((EB, SLAB), jnp.float32),
            pltpu.VMEM((EB, SLAB), jnp.float32),
            pltpu.VMEM_SHARED((NPAD, SLAB), jnp.float32),
            pltpu.SemaphoreType.DMA,
            pltpu.SemaphoreType.DMA,
            pltpu.SemaphoreType.DMA,
        ],
    )
    def seg(h_hbm, eif_hbm, ei_hbm, zeros_hbm, out_hbm, src_v, dst_v, rows0,
            rows1, agg_sh, sem0, sem1, zsem):
        cid = lax.axis_index("c")
        sid = lax.axis_index("s")
        rsl = pl.ds(sid * RPT, RPT)
        bufs = (rows0, rows1)
        sems = (sem0, sem1)

        def gstart(b, j):
            pltpu.async_copy(h_hbm.at[src_v.at[pl.ds(b * EB, EB)]],
                             bufs[j], sems[j])

        def gwait(b, j):
            pltpu.make_async_copy(h_hbm.at[src_v.at[pl.ds(b * EB, EB)]],
                                  bufs[j], sems[j]).wait()

        def scat(b, j):
            pltpu.sync_copy(bufs[j], agg_sh.at[dst_v.at[b]], add=True)

        pltpu.sync_copy(eif_hbm.at[pl.ds(sid * EPT, EPT)], src_v)
        pltpu.sync_copy(ei_hbm.at[1, sid], dst_v)
        for c in range(nchunk):
            zcp = pltpu.async_copy(zeros_hbm.at[rsl], agg_sh.at[rsl], zsem)
            def offs(j, _):
                sl = pl.ds(j * 16, 16)
                if c == 0:
                    src_v[sl] = src_v[sl] + cid * N
                else:
                    src_v[sl] = src_v[sl] + 2 * N
                return 0
            lax.fori_loop(0, EPT // 16, offs, 0)
            zcp.wait()
            plsc.subcore_barrier()
            gstart(0, 0)
            gstart(1, 1)

            def batch(k2, _):
                b0 = 2 * k2
                gwait(b0, 0)
                scat(b0, 0)
                gstart(b0 + 2, 0)
                gwait(b0 + 1, 1)
                scat(b0 + 1, 1)
                gstart(b0 + 3, 1)
                return 0
            lax.fori_loop(0, (NBATCH - 2) // 2, batch, 0)
            gwait(NBATCH - 2, 0)
            scat(NBATCH - 2, 0)
            gwait(NBATCH - 1, 1)
            scat(NBATCH - 1, 1)
            plsc.subcore_barrier()
            pltpu.sync_copy(agg_sh.at[rsl], out_hbm.at[c * 2 + cid, rsl])

    return seg


def _make_mlp(nslab_in: int):
    fin = nslab_in * SLAB
    R = 1000

    def body(h_ref, a_ref, wa_ref, ba_ref, wb_ref, bb_ref, o_ref):
        hh = jnp.concatenate(
            [h_ref[q] + a_ref[q] for q in range(nslab_in)], axis=1)
        t = jnp.dot(hh, wa_ref[...], preferred_element_type=jnp.float32)
        t = jnp.maximum(t + ba_ref[...], 0.0)
        o = jnp.dot(t, wb_ref[...], preferred_element_type=jnp.float32)
        o = jnp.maximum(o + bb_ref[...], 0.0)
        for q in range(4):
            o_ref[q] = o[:, q * SLAB:(q + 1) * SLAB]

    return pl.pallas_call(
        body,
        grid=(N // R,),
        in_specs=[
            pl.BlockSpec((nslab_in, R, SLAB), lambda i: (0, i, 0)),
            pl.BlockSpec((nslab_in, R, SLAB), lambda i: (0, i, 0)),
            pl.BlockSpec((fin, H), lambda i: (0, 0)),
            pl.BlockSpec((1, H), lambda i: (0, 0)),
            pl.BlockSpec((H, H), lambda i: (0, 0)),
            pl.BlockSpec((1, H), lambda i: (0, 0)),
        ],
        out_specs=pl.BlockSpec((4, R, SLAB), lambda i: (0, i, 0)),
        out_shape=jax.ShapeDtypeStruct((4, N, SLAB), jnp.float32),
    )


_RP = 1000


def _pool_body(b_ref, h_ref, w_ref, lb_ref, o_ref, sums_ref, cnt_ref):
    i = pl.program_id(0)

    @pl.when(i == 0)
    def _init():
        sums_ref[...] = jnp.zeros_like(sums_ref)
        cnt_ref[...] = jnp.zeros_like(cnt_ref)

    b = b_ref[0, 0, :]
    oh = (lax.broadcasted_iota(jnp.int32, (G, _RP), 0) == b[None, :]).astype(
        jnp.float32)
    h = jnp.concatenate([h_ref[q] for q in range(4)], axis=1)
    sums_ref[...] += jnp.dot(oh, h, preferred_element_type=jnp.float32)
    cnt_ref[...] += jnp.dot(oh, jnp.ones((_RP, 128), jnp.float32),
                            preferred_element_type=jnp.float32)

    @pl.when(i == pl.num_programs(0) - 1)
    def _fin():
        cnt = jnp.maximum(cnt_ref[:, 0:1], 1.0)
        pooled = sums_ref[...] / cnt
        o_ref[...] = jnp.dot(pooled, w_ref[...],
                             preferred_element_type=jnp.float32) + lb_ref[...]


_POOL = pl.pallas_call(
    _pool_body,
    grid=(N // _RP,),
    in_specs=[
        pl.BlockSpec((1, 1, _RP), lambda i: (i, 0, 0)),
        pl.BlockSpec((4, _RP, SLAB), lambda i: (0, i, 0)),
        pl.BlockSpec((H, 128), lambda i: (0, 0)),
        pl.BlockSpec((1, 128), lambda i: (0, 0)),
    ],
    out_specs=pl.BlockSpec((G, 128), lambda i: (0, 0)),
    out_shape=jax.ShapeDtypeStruct((G, 128), jnp.float32),
    scratch_shapes=[
        pltpu.VMEM((G, H), jnp.float32),
        pltpu.VMEM((G, 128), jnp.float32),
    ],
)

_SEG2 = _make_seg_sum(2)
_SEG4 = _make_seg_sum(4)
_MLP0 = _make_mlp(2)
_MLP1 = _make_mlp(4)


def kernel(x, edge_index, batch, W0a, b0a, W0b, b0b, W1a, b1a, W1b, b1b,
           W2a, b2a, W2b, b2b, linW, linb):
    zeros = jnp.zeros((NPAD, SLAB), jnp.float32)
    npad_e = E_PAD - E
    pad_src = jnp.arange(npad_e, dtype=jnp.int32) % N
    pad_dst = N + jnp.arange(npad_e, dtype=jnp.int32) % (NPAD - N)
    ep = jnp.concatenate([edge_index, jnp.stack([pad_src, pad_dst])], axis=1)
    ei = ep.reshape(2, NTILE, NBATCH, EB)
    srcf = ep[0]
    x_sm = x.reshape(N, 2, SLAB).transpose(1, 0, 2)
    agg0 = _SEG2(x_sm.reshape(2 * N, SLAB), srcf, ei, zeros)
    h1 = _MLP0(x_sm, agg0, W0a, b0a.reshape(1, H), W0b, b0b.reshape(1, H))
    agg1 = _SEG4(h1.reshape(4 * N, SLAB), srcf, ei, zeros)
    h2 = _MLP1(h1, agg1, W1a, b1a.reshape(1, H), W1b, b1b.reshape(1, H))
    agg2 = _SEG4(h2.reshape(4 * N, SLAB), srcf, ei, zeros)
    h3 = _MLP1(h2, agg2, W2a, b2a.reshape(1, H), W2b, b2b.reshape(1, H))
    wpad = jnp.zeros((H, 128), jnp.float32).at[:, :C].set(linW)
    bpad = jnp.zeros((1, 128), jnp.float32).at[0, :C].set(linb)
    out = _POOL(batch.reshape(N // _RP, 1, _RP), h3, wpad, bpad)
    return out[:, :C]


# speedup vs baseline: 6.5754x; 1.0062x over previous
import functools

import jax
import jax.numpy as jnp
from jax import lax
from jax.experimental import pallas as pl
from jax.experimental.pallas import tpu as pltpu
from jax.experimental.pallas import tpu_sc as plsc

N = 10000
E = 160000
F_IN = 256
H = 512
C = 40
G = 64

SLAB = 128
NTILE = 16
EB = 80
E_PAD = 163840
EPT = E_PAD // NTILE
NBATCH = EPT // EB
NPAD = 10240
RPT = NPAD // NTILE


def _make_seg_sum(nslab: int):
    nchunk = nslab // 2
    mesh = plsc.VectorSubcoreMesh(core_axis_name="c", subcore_axis_name="s",
                                  num_cores=2, num_subcores=NTILE)

    @functools.partial(
        pl.kernel,
        out_type=jax.ShapeDtypeStruct((nslab, NPAD, SLAB), jnp.float32),
        mesh=mesh,
        scratch_types=[
            pltpu.VMEM((EPT,), jnp.int32),
            pltpu.VMEM((NBATCH, EB), jnp.int32),
            pltpu.VMEM((EB, SLAB), jnp.float32),
            pltpu.VMEM((EB, SLAB), jnp.float32),
            pltpu.VMEM_SHARED((NPAD, SLAB), jnp.float32),
            pltpu.SemaphoreType.DMA,
            pltpu.SemaphoreType.DMA,
            pltpu.SemaphoreType.DMA,
        ],
    )
    def seg(h_hbm, srcq_hbm, ei_hbm, zeros_hbm, out_hbm, src_v, dst_v, rows0,
            rows1, agg_sh, sem0, sem1, zsem):
        cid = lax.axis_index("c")
        sid = lax.axis_index("s")
        rsl = pl.ds(sid * RPT, RPT)
        bufs = (rows0, rows1)
        sems = (sem0, sem1)

        def gstart(b, j):
            pltpu.async_copy(h_hbm.at[src_v.at[pl.ds(b * EB, EB)]],
                             bufs[j], sems[j])

        def gwait(b, j):
            pltpu.make_async_copy(h_hbm.at[src_v.at[pl.ds(b * EB, EB)]],
                                  bufs[j], sems[j]).wait()

        def scat(b, j):
            pltpu.sync_copy(bufs[j], agg_sh.at[dst_v.at[b]], add=True)

        pltpu.sync_copy(ei_hbm.at[1, sid], dst_v)
        for c in range(nchunk):
            q = c * 2 + cid
            icp = pltpu.async_copy(
                srcq_hbm.at[pl.ds(q * E_PAD + sid * EPT, EPT)], src_v, sem0)
            zcp = pltpu.async_copy(zeros_hbm.at[rsl], agg_sh.at[rsl], zsem)
            icp.wait()
            zcp.wait()
            plsc.subcore_barrier()
            gstart(0, 0)
            gstart(1, 1)

            def batch(k2, _):
                b0 = 2 * k2
                gwait(b0, 0)
                scat(b0, 0)
                gstart(b0 + 2, 0)
                gwait(b0 + 1, 1)
                scat(b0 + 1, 1)
                gstart(b0 + 3, 1)
                return 0
            lax.fori_loop(0, (NBATCH - 2) // 2, batch, 0)
            gwait(NBATCH - 2, 0)
            scat(NBATCH - 2, 0)
            gwait(NBATCH - 1, 1)
            scat(NBATCH - 1, 1)
            plsc.subcore_barrier()
            pltpu.sync_copy(agg_sh.at[rsl], out_hbm.at[c * 2 + cid, rsl])

    return seg


def _make_mlp(nslab_in: int):
    fin = nslab_in * SLAB
    R = 1000

    def body(h_ref, a_ref, wa_ref, ba_ref, wb_ref, bb_ref, o_ref):
        hh = jnp.concatenate(
            [h_ref[q] + a_ref[q] for q in range(nslab_in)], axis=1)
        t = jnp.dot(hh, wa_ref[...], preferred_element_type=jnp.float32)
        t = jnp.maximum(t + ba_ref[...], 0.0)
        o = jnp.dot(t, wb_ref[...], preferred_element_type=jnp.float32)
        o = jnp.maximum(o + bb_ref[...], 0.0)
        for q in range(4):
            o_ref[q] = o[:, q * SLAB:(q + 1) * SLAB]

    return pl.pallas_call(
        body,
        grid=(N // R,),
        in_specs=[
            pl.BlockSpec((nslab_in, R, SLAB), lambda i: (0, i, 0)),
            pl.BlockSpec((nslab_in, R, SLAB), lambda i: (0, i, 0)),
            pl.BlockSpec((fin, H), lambda i: (0, 0)),
            pl.BlockSpec((1, H), lambda i: (0, 0)),
            pl.BlockSpec((H, H), lambda i: (0, 0)),
            pl.BlockSpec((1, H), lambda i: (0, 0)),
        ],
        out_specs=pl.BlockSpec((4, R, SLAB), lambda i: (0, i, 0)),
        out_shape=jax.ShapeDtypeStruct((4, N, SLAB), jnp.float32),
    )


_RP = 1000


def _pool_body(b_ref, h_ref, w_ref, lb_ref, o_ref, sums_ref, cnt_ref):
    i = pl.program_id(0)

    @pl.when(i == 0)
    def _init():
        sums_ref[...] = jnp.zeros_like(sums_ref)
        cnt_ref[...] = jnp.zeros_like(cnt_ref)

    b = b_ref[0, 0, :]
    oh = (lax.broadcasted_iota(jnp.int32, (G, _RP), 0) == b[None, :]).astype(
        jnp.float32)
    h = jnp.concatenate([h_ref[q] for q in range(4)], axis=1)
    sums_ref[...] += jnp.dot(oh, h, preferred_element_type=jnp.float32)
    cnt_ref[...] += jnp.dot(oh, jnp.ones((_RP, 128), jnp.float32),
                            preferred_element_type=jnp.float32)

    @pl.when(i == pl.num_programs(0) - 1)
    def _fin():
        cnt = jnp.maximum(cnt_ref[:, 0:1], 1.0)
        pooled = sums_ref[...] / cnt
        o_ref[...] = jnp.dot(pooled, w_ref[...],
                             preferred_element_type=jnp.float32) + lb_ref[...]


_POOL = pl.pallas_call(
    _pool_body,
    grid=(N // _RP,),
    in_specs=[
        pl.BlockSpec((1, 1, _RP), lambda i: (i, 0, 0)),
        pl.BlockSpec((4, _RP, SLAB), lambda i: (0, i, 0)),
        pl.BlockSpec((H, 128), lambda i: (0, 0)),
        pl.BlockSpec((1, 128), lambda i: (0, 0)),
    ],
    out_specs=pl.BlockSpec((G, 128), lambda i: (0, 0)),
    out_shape=jax.ShapeDtypeStruct((G, 128), jnp.float32),
    scratch_shapes=[
        pltpu.VMEM((G, H), jnp.float32),
        pltpu.VMEM((G, 128), jnp.float32),
    ],
)

_SEG2 = _make_seg_sum(2)
_SEG4 = _make_seg_sum(4)
_MLP0 = _make_mlp(2)
_MLP1 = _make_mlp(4)


def kernel(x, edge_index, batch, W0a, b0a, W0b, b0b, W1a, b1a, W1b, b1b,
           W2a, b2a, W2b, b2b, linW, linb):
    zeros = jnp.zeros((NPAD, SLAB), jnp.float32)
    npad_e = E_PAD - E
    pad_src = jnp.arange(npad_e, dtype=jnp.int32) % N
    pad_dst = N + jnp.arange(npad_e, dtype=jnp.int32) % (NPAD - N)
    ep = jnp.concatenate([edge_index, jnp.stack([pad_src, pad_dst])], axis=1)
    ei = ep.reshape(2, NTILE, NBATCH, EB)
    qoff = jnp.arange(4, dtype=jnp.int32)[:, None] * N
    srcq4 = (ep[0][None, :] + qoff).reshape(-1)
    srcq2 = srcq4[:2 * E_PAD]
    x_sm = x.reshape(N, 2, SLAB).transpose(1, 0, 2)
    agg0 = _SEG2(x_sm.reshape(2 * N, SLAB), srcq2, ei, zeros)
    h1 = _MLP0(x_sm, agg0, W0a, b0a.reshape(1, H), W0b, b0b.reshape(1, H))
    agg1 = _SEG4(h1.reshape(4 * N, SLAB), srcq4, ei, zeros)
    h2 = _MLP1(h1, agg1, W1a, b1a.reshape(1, H), W1b, b1b.reshape(1, H))
    agg2 = _SEG4(h2.reshape(4 * N, SLAB), srcq4, ei, zeros)
    h3 = _MLP1(h2, agg2, W2a, b2a.reshape(1, H), W2b, b2b.reshape(1, H))
    wpad = jnp.zeros((H, 128), jnp.float32).at[:, :C].set(linW)
    bpad = jnp.zeros((1, 128), jnp.float32).at[0, :C].set(linb)
    out = _POOL(batch.reshape(N // _RP, 1, _RP), h3, wpad, bpad)
    return out[:, :C]


# speedup vs baseline: 7.0861x; 1.0777x over previous
import functools

import jax
import jax.numpy as jnp
from jax import lax
from jax.experimental import pallas as pl
from jax.experimental.pallas import tpu as pltpu
from jax.experimental.pallas import tpu_sc as plsc

N = 10000
E = 160000
F_IN = 256
H = 512
C = 40
G = 64

SLAB = 128
NTILE = 16
EB = 128
E_PAD = 163840
EPT = E_PAD // NTILE
NBATCH = EPT // EB
HB = NBATCH // 2
SRCH = HB * EB
NPAD = 10112
RPT = NPAD // NTILE


def _make_seg_sum(nslab: int):
    nchunk = nslab // 2
    mesh = plsc.VectorSubcoreMesh(core_axis_name="c", subcore_axis_name="s",
                                  num_cores=2, num_subcores=NTILE)

    @functools.partial(
        pl.kernel,
        out_type=jax.ShapeDtypeStruct((nslab, NPAD, SLAB), jnp.float32),
        mesh=mesh,
        scratch_types=[
            pltpu.VMEM((SRCH,), jnp.int32),
            pltpu.VMEM((NBATCH, EB), jnp.int32),
            pltpu.VMEM((EB, SLAB), jnp.float32),
            pltpu.VMEM((EB, SLAB), jnp.float32),
            pltpu.VMEM_SHARED((NPAD, SLAB), jnp.float32),
            pltpu.SemaphoreType.DMA,
            pltpu.SemaphoreType.DMA,
            pltpu.SemaphoreType.DMA,
            pltpu.SemaphoreType.DMA,
        ],
    )
    def seg(h_hbm, srcq_hbm, ei_hbm, zeros_hbm, out_hbm, src_v, dst_v, rows0,
            rows1, agg_sh, sem0, sem1, zsem, isem):
        cid = lax.axis_index("c")
        sid = lax.axis_index("s")
        rsl = pl.ds(sid * RPT, RPT)
        bufs = (rows0, rows1)
        sems = (sem0, sem1)

        def gstart(lb, j):
            pltpu.async_copy(h_hbm.at[src_v.at[pl.ds(lb * EB, EB)]],
                             bufs[j], sems[j])

        def gwait(lb, j):
            pltpu.make_async_copy(h_hbm.at[src_v.at[pl.ds(lb * EB, EB)]],
                                  bufs[j], sems[j]).wait()

        def scat(b, j):
            pltpu.sync_copy(bufs[j], agg_sh.at[dst_v.at[b]], add=True)

        pltpu.sync_copy(ei_hbm.at[1, sid], dst_v)
        for c in range(nchunk):
            ibase = (c * 2 + cid) * E_PAD + sid * EPT
            icp = pltpu.async_copy(
                srcq_hbm.at[pl.ds(ibase, SRCH)], src_v, isem)
            zcp = pltpu.async_copy(zeros_hbm.at[rsl], agg_sh.at[rsl], zsem)
            icp.wait()
            zcp.wait()
            plsc.subcore_barrier()
            for half in range(2):
                gb = half * HB
                gstart(0, 0)
                gstart(1, 1)

                def batch(k2, _, gb=gb):
                    lb0 = 2 * k2
                    gwait(lb0, 0)
                    scat(gb + lb0, 0)
                    gstart(lb0 + 2, 0)
                    gwait(lb0 + 1, 1)
                    scat(gb + lb0 + 1, 1)
                    gstart(lb0 + 3, 1)
                    return 0
                lax.fori_loop(0, (HB - 2) // 2, batch, 0)
                gwait(HB - 2, 0)
                scat(gb + HB - 2, 0)
                gwait(HB - 1, 1)
                scat(gb + HB - 1, 1)
                if half == 0:
                    pltpu.sync_copy(
                        srcq_hbm.at[pl.ds(ibase + SRCH, SRCH)], src_v)
            plsc.subcore_barrier()
            pltpu.sync_copy(agg_sh.at[rsl], out_hbm.at[c * 2 + cid, rsl])

    return seg


def _make_mlp(nslab_in: int):
    fin = nslab_in * SLAB
    R = 1000

    def body(h_ref, a_ref, wa_ref, ba_ref, wb_ref, bb_ref, o_ref):
        hh = jnp.concatenate(
            [h_ref[q] + a_ref[q] for q in range(nslab_in)], axis=1)
        t = jnp.dot(hh, wa_ref[...], preferred_element_type=jnp.float32)
        t = jnp.maximum(t + ba_ref[...], 0.0)
        o = jnp.dot(t, wb_ref[...], preferred_element_type=jnp.float32)
        o = jnp.maximum(o + bb_ref[...], 0.0)
        for q in range(4):
            o_ref[q] = o[:, q * SLAB:(q + 1) * SLAB]

    return pl.pallas_call(
        body,
        grid=(N // R,),
        in_specs=[
            pl.BlockSpec((nslab_in, R, SLAB), lambda i: (0, i, 0)),
            pl.BlockSpec((nslab_in, R, SLAB), lambda i: (0, i, 0)),
            pl.BlockSpec((fin, H), lambda i: (0, 0)),
            pl.BlockSpec((1, H), lambda i: (0, 0)),
            pl.BlockSpec((H, H), lambda i: (0, 0)),
            pl.BlockSpec((1, H), lambda i: (0, 0)),
        ],
        out_specs=pl.BlockSpec((4, R, SLAB), lambda i: (0, i, 0)),
        out_shape=jax.ShapeDtypeStruct((4, N, SLAB), jnp.float32),
    )


_RP = 1000


def _pool_body(b_ref, h_ref, w_ref, lb_ref, o_ref, sums_ref, cnt_ref):
    i = pl.program_id(0)

    @pl.when(i == 0)
    def _init():
        sums_ref[...] = jnp.zeros_like(sums_ref)
        cnt_ref[...] = jnp.zeros_like(cnt_ref)

    b = b_ref[0, 0, :]
    oh = (lax.broadcasted_iota(jnp.int32, (G, _RP), 0) == b[None, :]).astype(
        jnp.float32)
    h = jnp.concatenate([h_ref[q] for q in range(4)], axis=1)
    sums_ref[...] += jnp.dot(oh, h, preferred_element_type=jnp.float32)
    cnt_ref[...] += jnp.dot(oh, jnp.ones((_RP, 128), jnp.float32),
                            preferred_element_type=jnp.float32)

    @pl.when(i == pl.num_programs(0) - 1)
    def _fin():
        cnt = jnp.maximum(cnt_ref[:, 0:1], 1.0)
        pooled = sums_ref[...] / cnt
        o_ref[...] = jnp.dot(pooled, w_ref[...],
                             preferred_element_type=jnp.float32) + lb_ref[...]


_POOL = pl.pallas_call(
    _pool_body,
    grid=(N // _RP,),
    in_specs=[
        pl.BlockSpec((1, 1, _RP), lambda i: (i, 0, 0)),
        pl.BlockSpec((4, _RP, SLAB), lambda i: (0, i, 0)),
        pl.BlockSpec((H, 128), lambda i: (0, 0)),
        pl.BlockSpec((1, 128), lambda i: (0, 0)),
    ],
    out_specs=pl.BlockSpec((G, 128), lambda i: (0, 0)),
    out_shape=jax.ShapeDtypeStruct((G, 128), jnp.float32),
    scratch_shapes=[
        pltpu.VMEM((G, H), jnp.float32),
        pltpu.VMEM((G, 128), jnp.float32),
    ],
)

_SEG2 = _make_seg_sum(2)
_SEG4 = _make_seg_sum(4)
_MLP0 = _make_mlp(2)
_MLP1 = _make_mlp(4)


def kernel(x, edge_index, batch, W0a, b0a, W0b, b0b, W1a, b1a, W1b, b1b,
           W2a, b2a, W2b, b2b, linW, linb):
    zeros = jnp.zeros((NPAD, SLAB), jnp.float32)
    npad_e = E_PAD - E
    pad_src = jnp.arange(npad_e, dtype=jnp.int32) % N
    pad_dst = N + jnp.arange(npad_e, dtype=jnp.int32) % (NPAD - N)
    ep = jnp.concatenate([edge_index, jnp.stack([pad_src, pad_dst])], axis=1)
    ei = ep.reshape(2, NTILE, NBATCH, EB)
    qoff = jnp.arange(4, dtype=jnp.int32)[:, None] * N
    srcq4 = (ep[0][None, :] + qoff).reshape(-1)
    srcq2 = srcq4[:2 * E_PAD]
    x_sm = x.reshape(N, 2, SLAB).transpose(1, 0, 2)
    agg0 = _SEG2(x_sm.reshape(2 * N, SLAB), srcq2, ei, zeros)
    h1 = _MLP0(x_sm, agg0, W0a, b0a.reshape(1, H), W0b, b0b.reshape(1, H))
    agg1 = _SEG4(h1.reshape(4 * N, SLAB), srcq4, ei, zeros)
    h2 = _MLP1(h1, agg1, W1a, b1a.reshape(1, H), W1b, b1b.reshape(1, H))
    agg2 = _SEG4(h2.reshape(4 * N, SLAB), srcq4, ei, zeros)
    h3 = _MLP1(h2, agg2, W2a, b2a.reshape(1, H), W2b, b2b.reshape(1, H))
    wpad = jnp.zeros((H, 128), jnp.float32).at[:, :C].set(linW)
    bpad = jnp.zeros((1, 128), jnp.float32).at[0, :C].set(linb)
    out = _POOL(batch.reshape(N // _RP, 1, _RP), h3, wpad, bpad)
    return out[:, :C]


# speedup vs baseline: 7.0940x; 1.0011x over previous
import functools

import jax
import jax.numpy as jnp
from jax import lax
from jax.experimental import pallas as pl
from jax.experimental.pallas import tpu as pltpu
from jax.experimental.pallas import tpu_sc as plsc

N = 10000
E = 160000
F_IN = 256
H = 512
C = 40
G = 64

SLAB = 128
NTILE = 16
EB = 128
E_PAD = 163840
EPT = E_PAD // NTILE
NBATCH = EPT // EB
HB = NBATCH // 2
SRCH = HB * EB
NPAD = 10112
RPT = NPAD // NTILE


def _make_seg_sum(nslab: int):
    nchunk = nslab // 2
    mesh = plsc.VectorSubcoreMesh(core_axis_name="c", subcore_axis_name="s",
                                  num_cores=2, num_subcores=NTILE)

    @functools.partial(
        pl.kernel,
        out_type=jax.ShapeDtypeStruct((nslab, NPAD, SLAB), jnp.float32),
        mesh=mesh,
        scratch_types=[
            pltpu.VMEM((SRCH,), jnp.int32),
            pltpu.VMEM((NBATCH, EB), jnp.int32),
            pltpu.VMEM((EB, SLAB), jnp.float32),
            pltpu.VMEM((EB, SLAB), jnp.float32),
            pltpu.VMEM_SHARED((NPAD, SLAB), jnp.float32),
            pltpu.SemaphoreType.DMA,
            pltpu.SemaphoreType.DMA,
            pltpu.SemaphoreType.DMA,
            pltpu.SemaphoreType.DMA,
        ],
    )
    def seg(h_hbm, srcq_hbm, ei_hbm, zeros_hbm, out_hbm, src_v, dst_v, rows0,
            rows1, agg_sh, sem0, sem1, zsem, isem):
        cid = lax.axis_index("c")
        sid = lax.axis_index("s")
        rsl = pl.ds(sid * RPT, RPT)
        bufs = (rows0, rows1)
        sems = (sem0, sem1)

        def gstart(lb, j):
            pltpu.async_copy(h_hbm.at[src_v.at[pl.ds(lb * EB, EB)]],
                             bufs[j], sems[j])

        def gwait(lb, j):
            pltpu.make_async_copy(h_hbm.at[src_v.at[pl.ds(lb * EB, EB)]],
                                  bufs[j], sems[j]).wait()

        def scat(b, j):
            pltpu.sync_copy(bufs[j], agg_sh.at[dst_v.at[b]], add=True)

        pltpu.sync_copy(ei_hbm.at[1, sid], dst_v)
        for c in range(nchunk):
            ibase = (c * 2 + cid) * E_PAD + sid * EPT
            icp = pltpu.async_copy(
                srcq_hbm.at[pl.ds(ibase, SRCH)], src_v, isem)
            zcp = pltpu.async_copy(zeros_hbm.at[rsl], agg_sh.at[rsl], zsem)
            icp.wait()
            zcp.wait()
            plsc.subcore_barrier()
            for half in range(2):
                gb = half * HB
                gstart(0, 0)
                gstart(1, 1)

                def batch(k2, _, gb=gb):
                    lb0 = 2 * k2
                    gwait(lb0, 0)
                    scat(gb + lb0, 0)
                    gstart(lb0 + 2, 0)
                    gwait(lb0 + 1, 1)
                    scat(gb + lb0 + 1, 1)
                    gstart(lb0 + 3, 1)
                    return 0
                lax.fori_loop(0, (HB - 2) // 2, batch, 0)
                gwait(HB - 2, 0)
                scat(gb + HB - 2, 0)
                gwait(HB - 1, 1)
                scat(gb + HB - 1, 1)
                if half == 0:
                    pltpu.sync_copy(
                        srcq_hbm.at[pl.ds(ibase + SRCH, SRCH)], src_v)
            plsc.subcore_barrier()
            pltpu.sync_copy(agg_sh.at[rsl], out_hbm.at[c * 2 + cid, rsl])

    return seg


def _make_mlp(nslab_in: int):
    fin = nslab_in * SLAB
    R = 1000

    def body(h_ref, a_ref, wa_ref, ba_ref, wb_ref, bb_ref, o_ref):
        hh = jnp.concatenate(
            [h_ref[q] + a_ref[q] for q in range(nslab_in)], axis=1)
        t = jnp.dot(hh.astype(jnp.bfloat16), wa_ref[...].astype(jnp.bfloat16),
                    preferred_element_type=jnp.float32)
        t = jnp.maximum(t + ba_ref[...], 0.0)
        o = jnp.dot(t.astype(jnp.bfloat16), wb_ref[...].astype(jnp.bfloat16),
                    preferred_element_type=jnp.float32)
        o = jnp.maximum(o + bb_ref[...], 0.0)
        for q in range(4):
            o_ref[q] = o[:, q * SLAB:(q + 1) * SLAB]

    return pl.pallas_call(
        body,
        grid=(N // R,),
        in_specs=[
            pl.BlockSpec((nslab_in, R, SLAB), lambda i: (0, i, 0)),
            pl.BlockSpec((nslab_in, R, SLAB), lambda i: (0, i, 0)),
            pl.BlockSpec((fin, H), lambda i: (0, 0)),
            pl.BlockSpec((1, H), lambda i: (0, 0)),
            pl.BlockSpec((H, H), lambda i: (0, 0)),
            pl.BlockSpec((1, H), lambda i: (0, 0)),
        ],
        out_specs=pl.BlockSpec((4, R, SLAB), lambda i: (0, i, 0)),
        out_shape=jax.ShapeDtypeStruct((4, N, SLAB), jnp.float32),
    )


_RP = 1000


def _pool_body(b_ref, h_ref, w_ref, lb_ref, o_ref, sums_ref, cnt_ref):
    i = pl.program_id(0)

    @pl.when(i == 0)
    def _init():
        sums_ref[...] = jnp.zeros_like(sums_ref)
        cnt_ref[...] = jnp.zeros_like(cnt_ref)

    b = b_ref[0, 0, :]
    oh = (lax.broadcasted_iota(jnp.int32, (G, _RP), 0) == b[None, :]).astype(
        jnp.float32)
    h = jnp.concatenate([h_ref[q] for q in range(4)], axis=1)
    sums_ref[...] += jnp.dot(oh, h, preferred_element_type=jnp.float32)
    cnt_ref[...] += jnp.dot(oh, jnp.ones((_RP, 128), jnp.float32),
                            preferred_element_type=jnp.float32)

    @pl.when(i == pl.num_programs(0) - 1)
    def _fin():
        cnt = jnp.maximum(cnt_ref[:, 0:1], 1.0)
        pooled = sums_ref[...] / cnt
        o_ref[...] = jnp.dot(pooled, w_ref[...],
                             preferred_element_type=jnp.float32) + lb_ref[...]


_POOL = pl.pallas_call(
    _pool_body,
    grid=(N // _RP,),
    in_specs=[
        pl.BlockSpec((1, 1, _RP), lambda i: (i, 0, 0)),
        pl.BlockSpec((4, _RP, SLAB), lambda i: (0, i, 0)),
        pl.BlockSpec((H, 128), lambda i: (0, 0)),
        pl.BlockSpec((1, 128), lambda i: (0, 0)),
    ],
    out_specs=pl.BlockSpec((G, 128), lambda i: (0, 0)),
    out_shape=jax.ShapeDtypeStruct((G, 128), jnp.float32),
    scratch_shapes=[
        pltpu.VMEM((G, H), jnp.float32),
        pltpu.VMEM((G, 128), jnp.float32),
    ],
)

_SEG2 = _make_seg_sum(2)
_SEG4 = _make_seg_sum(4)
_MLP0 = _make_mlp(2)
_MLP1 = _make_mlp(4)


def kernel(x, edge_index, batch, W0a, b0a, W0b, b0b, W1a, b1a, W1b, b1b,
           W2a, b2a, W2b, b2b, linW, linb):
    zeros = jnp.zeros((NPAD, SLAB), jnp.float32)
    npad_e = E_PAD - E
    pad_src = jnp.arange(npad_e, dtype=jnp.int32) % N
    pad_dst = N + jnp.arange(npad_e, dtype=jnp.int32) % (NPAD - N)
    ep = jnp.concatenate([edge_index, jnp.stack([pad_src, pad_dst])], axis=1)
    ei = ep.reshape(2, NTILE, NBATCH, EB)
    qoff = jnp.arange(4, dtype=jnp.int32)[:, None] * N
    srcq4 = (ep[0][None, :] + qoff).reshape(-1)
    srcq2 = srcq4[:2 * E_PAD]
    x_sm = x.reshape(N, 2, SLAB).transpose(1, 0, 2)
    agg0 = _SEG2(x_sm.reshape(2 * N, SLAB), srcq2, ei, zeros)
    h1 = _MLP0(x_sm, agg0, W0a, b0a.reshape(1, H), W0b, b0b.reshape(1, H))
    agg1 = _SEG4(h1.reshape(4 * N, SLAB), srcq4, ei, zeros)
    h2 = _MLP1(h1, agg1, W1a, b1a.reshape(1, H), W1b, b1b.reshape(1, H))
    agg2 = _SEG4(h2.reshape(4 * N, SLAB), srcq4, ei, zeros)
    h3 = _MLP1(h2, agg2, W2a, b2a.reshape(1, H), W2b, b2b.reshape(1, H))
    wpad = jnp.zeros((H, 128), jnp.float32).at[:, :C].set(linW)
    bpad = jnp.zeros((1, 128), jnp.float32).at[0, :C].set(linb)
    out = _POOL(batch.reshape(N // _RP, 1, _RP), h3, wpad, bpad)
    return out[:, :C]


# speedup vs baseline: 7.1246x; 1.0043x over previous
import functools

import jax
import jax.numpy as jnp
from jax import lax
from jax.experimental import pallas as pl
from jax.experimental.pallas import tpu as pltpu
from jax.experimental.pallas import tpu_sc as plsc

N = 10000
E = 160000
F_IN = 256
H = 512
C = 40
G = 64

SLAB = 128
NTILE = 16
EB = 128
E_PAD = 163840
EPT = E_PAD // NTILE
NBATCH = EPT // EB
HB = NBATCH // 2
SRCH = HB * EB
NPAD = 10112
RPT = NPAD // NTILE


def _make_seg_sum(nslab: int):
    nchunk = nslab // 2
    mesh = plsc.VectorSubcoreMesh(core_axis_name="c", subcore_axis_name="s",
                                  num_cores=2, num_subcores=NTILE)

    @functools.partial(
        pl.kernel,
        out_type=jax.ShapeDtypeStruct((nslab, NPAD, SLAB), jnp.float32),
        mesh=mesh,
        scratch_types=[
            pltpu.VMEM((SRCH,), jnp.int32),
            pltpu.VMEM((NBATCH, EB), jnp.int32),
            pltpu.VMEM((EB, SLAB), jnp.float32),
            pltpu.VMEM((EB, SLAB), jnp.float32),
            pltpu.VMEM_SHARED((NPAD, SLAB), jnp.float32),
            pltpu.SemaphoreType.DMA,
            pltpu.SemaphoreType.DMA,
            pltpu.SemaphoreType.DMA,
            pltpu.SemaphoreType.DMA,
            pltpu.SemaphoreType.DMA,
        ],
    )
    def seg(h_hbm, srcq_hbm, ei_hbm, zeros_hbm, out_hbm, src_v, dst_v, rows0,
            rows1, agg_sh, sem0, sem1, zsem, isem, wsem):
        cid = lax.axis_index("c")
        sid = lax.axis_index("s")
        rsl = pl.ds(sid * RPT, RPT)
        bufs = (rows0, rows1)
        sems = (sem0, sem1)

        def gstart(lb, j):
            pltpu.async_copy(h_hbm.at[src_v.at[pl.ds(lb * EB, EB)]],
                             bufs[j], sems[j])

        def gwait(lb, j):
            pltpu.make_async_copy(h_hbm.at[src_v.at[pl.ds(lb * EB, EB)]],
                                  bufs[j], sems[j]).wait()

        def scat(b, j):
            pltpu.sync_copy(bufs[j], agg_sh.at[dst_v.at[b]], add=True)

        def ibase(c):
            return (c * 2 + cid) * E_PAD + sid * EPT

        pltpu.sync_copy(ei_hbm.at[1, sid], dst_v)
        icp = pltpu.async_copy(srcq_hbm.at[pl.ds(ibase(0), SRCH)], src_v, isem)
        zcp = pltpu.async_copy(zeros_hbm.at[rsl], agg_sh.at[rsl], zsem)
        icp.wait()
        zcp.wait()
        plsc.subcore_barrier()
        gstart(0, 0)
        gstart(1, 1)
        for c in range(nchunk):
            for half in range(2):
                gb = half * HB
                def batch(k2, _, gb=gb):
                    lb0 = 2 * k2
                    gwait(lb0, 0)
                    scat(gb + lb0, 0)
                    gstart(lb0 + 2, 0)
                    gwait(lb0 + 1, 1)
                    scat(gb + lb0 + 1, 1)
                    gstart(lb0 + 3, 1)
                    return 0
                lax.fori_loop(0, (HB - 2) // 2, batch, 0)
                gwait(HB - 2, 0)
                scat(gb + HB - 2, 0)
                gwait(HB - 1, 1)
                scat(gb + HB - 1, 1)
                if half == 0:
                    pltpu.sync_copy(
                        srcq_hbm.at[pl.ds(ibase(c) + SRCH, SRCH)], src_v)
                    gstart(0, 0)
                    gstart(1, 1)
            last = c == nchunk - 1
            if not last:
                icp = pltpu.async_copy(
                    srcq_hbm.at[pl.ds(ibase(c + 1), SRCH)], src_v, isem)
            plsc.subcore_barrier()
            wcp = pltpu.async_copy(agg_sh.at[rsl],
                                   out_hbm.at[c * 2 + cid, rsl], wsem)
            if not last:
                icp.wait()
                gstart(0, 0)
                gstart(1, 1)
                wcp.wait()
                zcp = pltpu.async_copy(zeros_hbm.at[rsl], agg_sh.at[rsl], zsem)
                zcp.wait()
                plsc.subcore_barrier()
            else:
                wcp.wait()

    return seg


def _make_mlp(nslab_in: int):
    fin = nslab_in * SLAB
    R = 1000

    def body(h_ref, a_ref, wa_ref, ba_ref, wb_ref, bb_ref, o_ref):
        hh = jnp.concatenate(
            [h_ref[q] + a_ref[q] for q in range(nslab_in)], axis=1)
        t = jnp.dot(hh, wa_ref[...], preferred_element_type=jnp.float32)
        t = jnp.maximum(t + ba_ref[...], 0.0)
        o = jnp.dot(t, wb_ref[...], preferred_element_type=jnp.float32)
        o = jnp.maximum(o + bb_ref[...], 0.0)
        for q in range(4):
            o_ref[q] = o[:, q * SLAB:(q + 1) * SLAB]

    return pl.pallas_call(
        body,
        grid=(N // R,),
        in_specs=[
            pl.BlockSpec((nslab_in, R, SLAB), lambda i: (0, i, 0)),
            pl.BlockSpec((nslab_in, R, SLAB), lambda i: (0, i, 0)),
            pl.BlockSpec((fin, H), lambda i: (0, 0)),
            pl.BlockSpec((1, H), lambda i: (0, 0)),
            pl.BlockSpec((H, H), lambda i: (0, 0)),
            pl.BlockSpec((1, H), lambda i: (0, 0)),
        ],
        out_specs=pl.BlockSpec((4, R, SLAB), lambda i: (0, i, 0)),
        out_shape=jax.ShapeDtypeStruct((4, N, SLAB), jnp.float32),
    )


_RP = 1000


def _pool_body(b_ref, h_ref, w_ref, lb_ref, o_ref, sums_ref, cnt_ref):
    i = pl.program_id(0)

    @pl.when(i == 0)
    def _init():
        sums_ref[...] = jnp.zeros_like(sums_ref)
        cnt_ref[...] = jnp.zeros_like(cnt_ref)

    b = b_ref[0, 0, :]
    oh = (lax.broadcasted_iota(jnp.int32, (G, _RP), 0) == b[None, :]).astype(
        jnp.float32)
    h = jnp.concatenate([h_ref[q] for q in range(4)], axis=1)
    sums_ref[...] += jnp.dot(oh, h, preferred_element_type=jnp.float32)
    cnt_ref[...] += jnp.dot(oh, jnp.ones((_RP, 128), jnp.float32),
                            preferred_element_type=jnp.float32)

    @pl.when(i == pl.num_programs(0) - 1)
    def _fin():
        cnt = jnp.maximum(cnt_ref[:, 0:1], 1.0)
        pooled = sums_ref[...] / cnt
        o_ref[...] = jnp.dot(pooled, w_ref[...],
                             preferred_element_type=jnp.float32) + lb_ref[...]


_POOL = pl.pallas_call(
    _pool_body,
    grid=(N // _RP,),
    in_specs=[
        pl.BlockSpec((1, 1, _RP), lambda i: (i, 0, 0)),
        pl.BlockSpec((4, _RP, SLAB), lambda i: (0, i, 0)),
        pl.BlockSpec((H, 128), lambda i: (0, 0)),
        pl.BlockSpec((1, 128), lambda i: (0, 0)),
    ],
    out_specs=pl.BlockSpec((G, 128), lambda i: (0, 0)),
    out_shape=jax.ShapeDtypeStruct((G, 128), jnp.float32),
    scratch_shapes=[
        pltpu.VMEM((G, H), jnp.float32),
        pltpu.VMEM((G, 128), jnp.float32),
    ],
)

_SEG2 = _make_seg_sum(2)
_SEG4 = _make_seg_sum(4)
_MLP0 = _make_mlp(2)
_MLP1 = _make_mlp(4)


def kernel(x, edge_index, batch, W0a, b0a, W0b, b0b, W1a, b1a, W1b, b1b,
           W2a, b2a, W2b, b2b, linW, linb):
    zeros = jnp.zeros((NPAD, SLAB), jnp.float32)
    npad_e = E_PAD - E
    pad_src = jnp.arange(npad_e, dtype=jnp.int32) % N
    pad_dst = N + jnp.arange(npad_e, dtype=jnp.int32) % (NPAD - N)
    ep = jnp.concatenate([edge_index, jnp.stack([pad_src, pad_dst])], axis=1)
    ei = ep.reshape(2, NTILE, NBATCH, EB)
    qoff = jnp.arange(4, dtype=jnp.int32)[:, None] * N
    srcq4 = (ep[0][None, :] + qoff).reshape(-1)
    srcq2 = srcq4[:2 * E_PAD]
    x_sm = x.reshape(N, 2, SLAB).transpose(1, 0, 2)
    agg0 = _SEG2(x_sm.reshape(2 * N, SLAB), srcq2, ei, zeros)
    h1 = _MLP0(x_sm, agg0, W0a, b0a.reshape(1, H), W0b, b0b.reshape(1, H))
    agg1 = _SEG4(h1.reshape(4 * N, SLAB), srcq4, ei, zeros)
    h2 = _MLP1(h1, agg1, W1a, b1a.reshape(1, H), W1b, b1b.reshape(1, H))
    agg2 = _SEG4(h2.reshape(4 * N, SLAB), srcq4, ei, zeros)
    h3 = _MLP1(h2, agg2, W2a, b2a.reshape(1, H), W2b, b2b.reshape(1, H))
    wpad = jnp.zeros((H, 128), jnp.float32).at[:, :C].set(linW)
    bpad = jnp.zeros((1, 128), jnp.float32).at[0, :C].set(linb)
    out = _POOL(batch.reshape(N // _RP, 1, _RP), h3, wpad, bpad)
    return out[:, :C]


# speedup vs baseline: 7.6328x; 1.0713x over previous
import functools

import jax
import jax.numpy as jnp
from jax import lax
from jax.experimental import pallas as pl
from jax.experimental.pallas import tpu as pltpu
from jax.experimental.pallas import tpu_sc as plsc

N = 10000
E = 160000
F_IN = 256
H = 512
C = 40
G = 64

SLAB = 128
NTILE = 16
EB = 80
E_PAD = 163840
EPT = E_PAD // NTILE
NBATCH = EPT // EB
HB = NBATCH // 2
SRCH = HB * EB
NPAD = 10112
RPT = NPAD // NTILE


def _make_seg_sum(nslab: int):
    nchunk = nslab // 2
    mesh = plsc.VectorSubcoreMesh(core_axis_name="c", subcore_axis_name="s",
                                  num_cores=2, num_subcores=NTILE)

    @functools.partial(
        pl.kernel,
        out_type=jax.ShapeDtypeStruct((nslab, NPAD, SLAB), jnp.float32),
        mesh=mesh,
        scratch_types=[
            pltpu.VMEM((SRCH,), jnp.int32),
            pltpu.VMEM((HB, EB), jnp.int32),
            pltpu.VMEM((EB, SLAB), jnp.float32),
            pltpu.VMEM((EB, SLAB), jnp.float32),
            pltpu.VMEM((EB, SLAB), jnp.float32),
            pltpu.VMEM_SHARED((NPAD, SLAB), jnp.float32),
            pltpu.SemaphoreType.DMA,
            pltpu.SemaphoreType.DMA,
            pltpu.SemaphoreType.DMA,
            pltpu.SemaphoreType.DMA,
            pltpu.SemaphoreType.DMA,
            pltpu.SemaphoreType.DMA,
            pltpu.SemaphoreType.DMA,
        ],
    )
    def seg(h_hbm, srcq_hbm, ei_hbm, zeros_hbm, out_hbm, src_v, dst_v, rows0,
            rows1, rows2, agg_sh, sem0, sem1, sem2, zsem, isem, wsem, dsem):
        cid = lax.axis_index("c")
        sid = lax.axis_index("s")
        rsl = pl.ds(sid * RPT, RPT)
        bufs = (rows0, rows1, rows2)
        sems = (sem0, sem1, sem2)

        def gstart(lb, j):
            pltpu.async_copy(h_hbm.at[src_v.at[pl.ds(lb * EB, EB)]],
                             bufs[j], sems[j])

        def gwait(lb, j):
            pltpu.make_async_copy(h_hbm.at[src_v.at[pl.ds(lb * EB, EB)]],
                                  bufs[j], sems[j]).wait()

        def scat(b, j):
            pltpu.sync_copy(bufs[j], agg_sh.at[dst_v.at[b]], add=True)

        def ibase(c):
            return (c * 2 + cid) * E_PAD + sid * EPT

        def prime():
            gstart(0, 0)
            gstart(1, 1)
            gstart(2, 2)

        pltpu.sync_copy(ei_hbm.at[1, sid, pl.ds(0, HB)], dst_v)
        icp = pltpu.async_copy(srcq_hbm.at[pl.ds(ibase(0), SRCH)], src_v, isem)
        zcp = pltpu.async_copy(zeros_hbm.at[rsl], agg_sh.at[rsl], zsem)
        icp.wait()
        zcp.wait()
        plsc.subcore_barrier()
        prime()
        for c in range(nchunk):
            for half in range(2):
                def batch(k3, _):
                    b0 = 3 * k3
                    for j in range(3):
                        gwait(b0 + j, j)
                        scat(b0 + j, j)
                        gstart(b0 + 3 + j, j)
                    return 0
                lax.fori_loop(0, (HB - 4) // 3, batch, 0)
                gwait(HB - 4, 0)
                scat(HB - 4, 0)
                gstart(HB - 1, 0)
                gwait(HB - 3, 1)
                scat(HB - 3, 1)
                gwait(HB - 2, 2)
                scat(HB - 2, 2)
                gwait(HB - 1, 0)
                scat(HB - 1, 0)
                if half == 0:
                    pltpu.sync_copy(
                        srcq_hbm.at[pl.ds(ibase(c) + SRCH, SRCH)], src_v)
                    pltpu.sync_copy(ei_hbm.at[1, sid, pl.ds(HB, HB)], dst_v)
                    prime()
            last = c == nchunk - 1
            if not last:
                icp = pltpu.async_copy(
                    srcq_hbm.at[pl.ds(ibase(c + 1), SRCH)], src_v, isem)
                dcp = pltpu.async_copy(ei_hbm.at[1, sid, pl.ds(0, HB)],
                                       dst_v, dsem)
            plsc.subcore_barrier()
            wcp = pltpu.async_copy(agg_sh.at[rsl],
                                   out_hbm.at[c * 2 + cid, rsl], wsem)
            if not last:
                icp.wait()
                prime()
                dcp.wait()
                wcp.wait()
                zcp = pltpu.async_copy(zeros_hbm.at[rsl], agg_sh.at[rsl], zsem)
                zcp.wait()
                plsc.subcore_barrier()
            else:
                wcp.wait()

    return seg


def _make_mlp(nslab_in: int):
    fin = nslab_in * SLAB
    R = 1000

    def body(h_ref, a_ref, wa_ref, ba_ref, wb_ref, bb_ref, o_ref):
        hh = jnp.concatenate(
            [h_ref[q] + a_ref[q] for q in range(nslab_in)], axis=1)
        t = jnp.dot(hh, wa_ref[...], preferred_element_type=jnp.float32)
        t = jnp.maximum(t + ba_ref[...], 0.0)
        o = jnp.dot(t, wb_ref[...], preferred_element_type=jnp.float32)
        o = jnp.maximum(o + bb_ref[...], 0.0)
        for q in range(4):
            o_ref[q] = o[:, q * SLAB:(q + 1) * SLAB]

    return pl.pallas_call(
        body,
        grid=(N // R,),
        in_specs=[
            pl.BlockSpec((nslab_in, R, SLAB), lambda i: (0, i, 0)),
            pl.BlockSpec((nslab_in, R, SLAB), lambda i: (0, i, 0)),
            pl.BlockSpec((fin, H), lambda i: (0, 0)),
            pl.BlockSpec((1, H), lambda i: (0, 0)),
            pl.BlockSpec((H, H), lambda i: (0, 0)),
            pl.BlockSpec((1, H), lambda i: (0, 0)),
        ],
        out_specs=pl.BlockSpec((4, R, SLAB), lambda i: (0, i, 0)),
        out_shape=jax.ShapeDtypeStruct((4, N, SLAB), jnp.float32),
    )


_RP = 1000


def _pool_body(b_ref, h_ref, w_ref, lb_ref, o_ref, sums_ref, cnt_ref):
    i = pl.program_id(0)

    @pl.when(i == 0)
    def _init():
        sums_ref[...] = jnp.zeros_like(sums_ref)
        cnt_ref[...] = jnp.zeros_like(cnt_ref)

    b = b_ref[0, 0, :]
    oh = (lax.broadcasted_iota(jnp.int32, (G, _RP), 0) == b[None, :]).astype(
        jnp.float32)
    h = jnp.concatenate([h_ref[q] for q in range(4)], axis=1)
    sums_ref[...] += jnp.dot(oh, h, preferred_element_type=jnp.float32)
    cnt_ref[...] += jnp.dot(oh, jnp.ones((_RP, 128), jnp.float32),
                            preferred_element_type=jnp.float32)

    @pl.when(i == pl.num_programs(0) - 1)
    def _fin():
        cnt = jnp.maximum(cnt_ref[:, 0:1], 1.0)
        pooled = sums_ref[...] / cnt
        o_ref[...] = jnp.dot(pooled, w_ref[...],
                             preferred_element_type=jnp.float32) + lb_ref[...]


_POOL = pl.pallas_call(
    _pool_body,
    grid=(N // _RP,),
    in_specs=[
        pl.BlockSpec((1, 1, _RP), lambda i: (i, 0, 0)),
        pl.BlockSpec((4, _RP, SLAB), lambda i: (0, i, 0)),
        pl.BlockSpec((H, 128), lambda i: (0, 0)),
        pl.BlockSpec((1, 128), lambda i: (0, 0)),
    ],
    out_specs=pl.BlockSpec((G, 128), lambda i: (0, 0)),
    out_shape=jax.ShapeDtypeStruct((G, 128), jnp.float32),
    scratch_shapes=[
        pltpu.VMEM((G, H), jnp.float32),
        pltpu.VMEM((G, 128), jnp.float32),
    ],
)

_SEG2 = _make_seg_sum(2)
_SEG4 = _make_seg_sum(4)
_MLP0 = _make_mlp(2)
_MLP1 = _make_mlp(4)


def kernel(x, edge_index, batch, W0a, b0a, W0b, b0b, W1a, b1a, W1b, b1b,
           W2a, b2a, W2b, b2b, linW, linb):
    zeros = jnp.zeros((NPAD, SLAB), jnp.float32)
    npad_e = E_PAD - E
    pad_src = jnp.arange(npad_e, dtype=jnp.int32) % N
    pad_dst = N + jnp.arange(npad_e, dtype=jnp.int32) % (NPAD - N)
    ep = jnp.concatenate([edge_index, jnp.stack([pad_src, pad_dst])], axis=1)
    ei = ep.reshape(2, NTILE, NBATCH, EB)
    qoff = jnp.arange(4, dtype=jnp.int32)[:, None] * N
    srcq4 = (ep[0][None, :] + qoff).reshape(-1)
    srcq2 = srcq4[:2 * E_PAD]
    x_sm = x.reshape(N, 2, SLAB).transpose(1, 0, 2)
    agg0 = _SEG2(x_sm.reshape(2 * N, SLAB), srcq2, ei, zeros)
    h1 = _MLP0(x_sm, agg0, W0a, b0a.reshape(1, H), W0b, b0b.reshape(1, H))
    agg1 = _SEG4(h1.reshape(4 * N, SLAB), srcq4, ei, zeros)
    h2 = _MLP1(h1, agg1, W1a, b1a.reshape(1, H), W1b, b1b.reshape(1, H))
    agg2 = _SEG4(h2.reshape(4 * N, SLAB), srcq4, ei, zeros)
    h3 = _MLP1(h2, agg2, W2a, b2a.reshape(1, H), W2b, b2b.reshape(1, H))
    wpad = jnp.zeros((H, 128), jnp.float32).at[:, :C].set(linW)
    bpad = jnp.zeros((1, 128), jnp.float32).at[0, :C].set(linb)
    out = _POOL(batch.reshape(N // _RP, 1, _RP), h3, wpad, bpad)
    return out[:, :C]


# speedup vs baseline: 7.6912x; 1.0076x over previous
import functools

import jax
import jax.numpy as jnp
from jax import lax
from jax.experimental import pallas as pl
from jax.experimental.pallas import tpu as pltpu
from jax.experimental.pallas import tpu_sc as plsc

N = 10000
E = 160000
F_IN = 256
H = 512
C = 40
G = 64

SLAB = 128
NTILE = 16
EB = 80
E_PAD = 163840
EPT = E_PAD // NTILE
NBATCH = EPT // EB
HB = NBATCH // 2
SRCH = HB * EB
NPAD = 10112
RPT = NPAD // NTILE


def _make_seg_sum(nslab: int):
    nchunk = nslab // 2
    mesh = plsc.VectorSubcoreMesh(core_axis_name="c", subcore_axis_name="s",
                                  num_cores=2, num_subcores=NTILE)

    @functools.partial(
        pl.kernel,
        out_type=jax.ShapeDtypeStruct((nslab, NPAD, SLAB), jnp.float32),
        mesh=mesh,
        scratch_types=[
            pltpu.VMEM((SRCH,), jnp.int32),
            pltpu.VMEM((HB, EB), jnp.int32),
            pltpu.VMEM((EB, SLAB), jnp.float32),
            pltpu.VMEM((EB, SLAB), jnp.float32),
            pltpu.VMEM((EB, SLAB), jnp.float32),
            pltpu.VMEM_SHARED((NPAD, SLAB), jnp.float32),
            pltpu.SemaphoreType.DMA,
            pltpu.SemaphoreType.DMA,
            pltpu.SemaphoreType.DMA,
            pltpu.SemaphoreType.DMA,
            pltpu.SemaphoreType.DMA,
            pltpu.SemaphoreType.DMA,
            pltpu.SemaphoreType.DMA,
        ],
    )
    def seg(h_hbm, srcq_hbm, ei_hbm, zeros_hbm, out_hbm, src_v, dst_v, rows0,
            rows1, rows2, agg_sh, sem0, sem1, sem2, zsem, isem, wsem, dsem):
        cid = lax.axis_index("c")
        sid = lax.axis_index("s")
        rsl = pl.ds(sid * RPT, RPT)
        bufs = (rows0, rows1, rows2)
        sems = (sem0, sem1, sem2)

        def gstart(lb, j):
            pltpu.async_copy(h_hbm.at[src_v.at[pl.ds(lb * EB, EB)]],
                             bufs[j], sems[j])

        def gwait(lb, j):
            pltpu.make_async_copy(h_hbm.at[src_v.at[pl.ds(lb * EB, EB)]],
                                  bufs[j], sems[j]).wait()

        def scat(b, j):
            pltpu.sync_copy(bufs[j], agg_sh.at[dst_v.at[b]], add=True)

        def ibase(c):
            return (c * 2 + cid) * E_PAD + sid * EPT

        def prime():
            gstart(0, 0)
            gstart(1, 1)
            gstart(2, 2)

        pltpu.sync_copy(ei_hbm.at[1, sid, pl.ds(0, HB)], dst_v)
        icp = pltpu.async_copy(srcq_hbm.at[pl.ds(ibase(0), SRCH)], src_v, isem)
        zcp = pltpu.async_copy(zeros_hbm.at[rsl], agg_sh.at[rsl], zsem)
        icp.wait()
        zcp.wait()
        plsc.subcore_barrier()
        prime()
        for c in range(nchunk):
            for half in range(2):
                def batch(k3, _):
                    b0 = 3 * k3
                    for j in range(3):
                        gwait(b0 + j, j)
                        scat(b0 + j, j)
                        gstart(b0 + 3 + j, j)
                    return 0
                lax.fori_loop(0, (HB - 4) // 3, batch, 0)
                gwait(HB - 4, 0)
                scat(HB - 4, 0)
                gstart(HB - 1, 0)
                gwait(HB - 3, 1)
                scat(HB - 3, 1)
                gwait(HB - 2, 2)
                scat(HB - 2, 2)
                gwait(HB - 1, 0)
                scat(HB - 1, 0)
                if half == 0:
                    pltpu.sync_copy(
                        srcq_hbm.at[pl.ds(ibase(c) + SRCH, SRCH)], src_v)
                    pltpu.sync_copy(ei_hbm.at[1, sid, pl.ds(HB, HB)], dst_v)
                    prime()
            last = c == nchunk - 1
            if not last:
                icp = pltpu.async_copy(
                    srcq_hbm.at[pl.ds(ibase(c + 1), SRCH)], src_v, isem)
                dcp = pltpu.async_copy(ei_hbm.at[1, sid, pl.ds(0, HB)],
                                       dst_v, dsem)
            plsc.subcore_barrier()
            wcp = pltpu.async_copy(agg_sh.at[rsl],
                                   out_hbm.at[c * 2 + cid, rsl], wsem)
            if not last:
                icp.wait()
                prime()
                dcp.wait()
                wcp.wait()
                zcp = pltpu.async_copy(zeros_hbm.at[rsl], agg_sh.at[rsl], zsem)
                zcp.wait()
                plsc.subcore_barrier()
            else:
                wcp.wait()

    return seg


def _make_mlp(nslab_in: int):
    fin = nslab_in * SLAB
    R = 2000

    def body(h_ref, a_ref, wa_ref, ba_ref, wb_ref, bb_ref, o_ref):
        hh = jnp.concatenate(
            [h_ref[q] + a_ref[q] for q in range(nslab_in)], axis=1)
        t = jnp.dot(hh, wa_ref[...], preferred_element_type=jnp.float32)
        t = jnp.maximum(t + ba_ref[...], 0.0)
        o = jnp.dot(t, wb_ref[...], preferred_element_type=jnp.float32)
        o = jnp.maximum(o + bb_ref[...], 0.0)
        for q in range(4):
            o_ref[q] = o[:, q * SLAB:(q + 1) * SLAB]

    return pl.pallas_call(
        body,
        grid=(N // R,),
        in_specs=[
            pl.BlockSpec((nslab_in, R, SLAB), lambda i: (0, i, 0)),
            pl.BlockSpec((nslab_in, R, SLAB), lambda i: (0, i, 0)),
            pl.BlockSpec((fin, H), lambda i: (0, 0)),
            pl.BlockSpec((1, H), lambda i: (0, 0)),
            pl.BlockSpec((H, H), lambda i: (0, 0)),
            pl.BlockSpec((1, H), lambda i: (0, 0)),
        ],
        out_specs=pl.BlockSpec((4, R, SLAB), lambda i: (0, i, 0)),
        out_shape=jax.ShapeDtypeStruct((4, N, SLAB), jnp.float32),
    )


_RP = 2000


def _pool_body(b_ref, h_ref, w_ref, lb_ref, o_ref, sums_ref, cnt_ref):
    i = pl.program_id(0)

    @pl.when(i == 0)
    def _init():
        sums_ref[...] = jnp.zeros_like(sums_ref)
        cnt_ref[...] = jnp.zeros_like(cnt_ref)

    b = b_ref[0, 0, :]
    oh = (lax.broadcasted_iota(jnp.int32, (G, _RP), 0) == b[None, :]).astype(
        jnp.float32)
    h = jnp.concatenate([h_ref[q] for q in range(4)], axis=1)
    sums_ref[...] += jnp.dot(oh, h, preferred_element_type=jnp.float32)
    cnt_ref[...] += jnp.dot(oh, jnp.ones((_RP, 128), jnp.float32),
                            preferred_element_type=jnp.float32)

    @pl.when(i == pl.num_programs(0) - 1)
    def _fin():
        cnt = jnp.maximum(cnt_ref[:, 0:1], 1.0)
        pooled = sums_ref[...] / cnt
        o_ref[...] = jnp.dot(pooled, w_ref[...],
                             preferred_element_type=jnp.float32) + lb_ref[...]


_POOL = pl.pallas_call(
    _pool_body,
    grid=(N // _RP,),
    in_specs=[
        pl.BlockSpec((1, 1, _RP), lambda i: (i, 0, 0)),
        pl.BlockSpec((4, _RP, SLAB), lambda i: (0, i, 0)),
        pl.BlockSpec((H, 128), lambda i: (0, 0)),
        pl.BlockSpec((1, 128), lambda i: (0, 0)),
    ],
    out_specs=pl.BlockSpec((G, 128), lambda i: (0, 0)),
    out_shape=jax.ShapeDtypeStruct((G, 128), jnp.float32),
    scratch_shapes=[
        pltpu.VMEM((G, H), jnp.float32),
        pltpu.VMEM((G, 128), jnp.float32),
    ],
)

_SEG2 = _make_seg_sum(2)
_SEG4 = _make_seg_sum(4)
_MLP0 = _make_mlp(2)
_MLP1 = _make_mlp(4)


def kernel(x, edge_index, batch, W0a, b0a, W0b, b0b, W1a, b1a, W1b, b1b,
           W2a, b2a, W2b, b2b, linW, linb):
    zeros = jnp.zeros((NPAD, SLAB), jnp.float32)
    npad_e = E_PAD - E
    pad_src = jnp.arange(npad_e, dtype=jnp.int32) % N
    pad_dst = N + jnp.arange(npad_e, dtype=jnp.int32) % (NPAD - N)
    ep = jnp.concatenate([edge_index, jnp.stack([pad_src, pad_dst])], axis=1)
    ei = ep.reshape(2, NTILE, NBATCH, EB)
    qoff = jnp.arange(4, dtype=jnp.int32)[:, None] * N
    srcq4 = (ep[0][None, :] + qoff).reshape(-1)
    srcq2 = srcq4[:2 * E_PAD]
    x_sm = x.reshape(N, 2, SLAB).transpose(1, 0, 2)
    agg0 = _SEG2(x_sm.reshape(2 * N, SLAB), srcq2, ei, zeros)
    h1 = _MLP0(x_sm, agg0, W0a, b0a.reshape(1, H), W0b, b0b.reshape(1, H))
    agg1 = _SEG4(h1.reshape(4 * N, SLAB), srcq4, ei, zeros)
    h2 = _MLP1(h1, agg1, W1a, b1a.reshape(1, H), W1b, b1b.reshape(1, H))
    agg2 = _SEG4(h2.reshape(4 * N, SLAB), srcq4, ei, zeros)
    h3 = _MLP1(h2, agg2, W2a, b2a.reshape(1, H), W2b, b2b.reshape(1, H))
    wpad = jnp.zeros((H, 128), jnp.float32).at[:, :C].set(linW)
    bpad = jnp.zeros((1, 128), jnp.float32).at[0, :C].set(linb)
    out = _POOL(batch.reshape(N // _RP, 1, _RP), h3, wpad, bpad)
    return out[:, :C]
